# Initial kernel scaffold; baseline (speedup 1.0000x reference)
import jax
import jax.numpy as jnp
from jax.experimental import pallas as pl


def kernel(users, movies, user_table, movie_table, W1, b1, W2, b2, W3, b3, W4, b4, W5, b5, g1, be1, g2, be2, g3, be3, g4, be4):
    raise NotImplementedError("write your pallas kernel here")


# speedup vs baseline: 2.1195x; 2.1195x over previous
import functools
import math

import jax
import jax.numpy as jnp
from jax import lax
from jax.experimental import pallas as pl
from jax.experimental.pallas import tpu as pltpu
from jax.experimental.pallas import tpu_sc as plsc

_B = 16384
_D = 128
_EPS = 1e-5
_INV = 1.0 / math.sqrt(1.0 + _EPS)
_T = 1024
_NT = _B // _T
_CH = 128


@functools.cache
def _gather_kernel():
    info = plsc.get_sparse_core_info()
    nc, ns = info.num_cores, info.num_subcores
    nw = nc * ns
    b_per_w = _B // nw
    n_ch = b_per_w // _CH
    mesh = plsc.VectorSubcoreMesh(core_axis_name="c", subcore_axis_name="s")

    @functools.partial(
        pl.kernel,
        mesh=mesh,
        out_type=[
            jax.ShapeDtypeStruct((_B, _D), jnp.float32),
            jax.ShapeDtypeStruct((_B, _D), jnp.float32),
        ],
        scratch_types=[
            pltpu.VMEM((_CH,), jnp.int32),
            pltpu.VMEM((_CH, _D), jnp.float32),
            pltpu.SemaphoreType.DMA,
        ],
    )
    def gather(ut, mt, users, movies, ue, me, idx_v, rows_v, sem):
        wid = lax.axis_index("s") * nc + lax.axis_index("c")
        base = wid * b_per_w
        for table, idx_hbm, out_hbm in ((ut, users, ue), (mt, movies, me)):
            for j in range(n_ch):
                off = base + j * _CH
                pltpu.sync_copy(idx_hbm.at[pl.ds(off, _CH)], idx_v)
                pltpu.async_copy(table.at[idx_v], rows_v, sem).wait()
                pltpu.sync_copy(rows_v, out_hbm.at[pl.ds(off, _CH)])

    return gather


def _mlp_body(ue, me, W1, W2, W3, W4, W5,
              b1, b2, b3, b4, b5, g1, be1, g2, be2, g3, be3, g4, be4, out):
    f32 = jnp.float32
    h = (jnp.dot(ue[...], W1[0:_D, :], preferred_element_type=f32)
         + jnp.dot(me[...], W1[_D:2 * _D, :], preferred_element_type=f32)
         + b1[...])
    h = jnp.maximum(h, 0.0) * (g1[...] * _INV) + be1[...]
    h = jnp.dot(h, W2[...], preferred_element_type=f32) + b2[...]
    h = jnp.maximum(h, 0.0) * (g2[...] * _INV) + be2[...]
    h = jnp.dot(h, W3[...], preferred_element_type=f32) + b3[...]
    h = jnp.maximum(h, 0.0) * (g3[...] * _INV) + be3[...]
    h = jnp.dot(h, W4[...], preferred_element_type=f32) + b4[...]
    h = jnp.maximum(h, 0.0) * (g4[...] * _INV) + be4[...]
    h = jnp.dot(h, W5[...], preferred_element_type=f32) + b5[...]
    out[...] = 1.0 / (1.0 + jnp.exp(-h))


def _mlp(ue, me, W1, W2, W3, W4, W5, b1, b2, b3, b4, b5,
         g1, be1, g2, be2, g3, be3, g4, be4):
    full = lambda shape: pl.BlockSpec(shape, lambda i: (0, 0))
    return pl.pallas_call(
        _mlp_body,
        grid=(_NT,),
        in_specs=[
            pl.BlockSpec((_T, _D), lambda i: (i, 0)),
            pl.BlockSpec((_T, _D), lambda i: (i, 0)),
            full((2 * _D, 512)), full((512, 256)), full((256, 128)),
            full((128, 64)), full((64, 1)),
            full((1, 512)), full((1, 256)), full((1, 128)), full((1, 64)),
            full((1, 1)),
            full((1, 512)), full((1, 512)),
            full((1, 256)), full((1, 256)),
            full((1, 128)), full((1, 128)),
            full((1, 64)), full((1, 64)),
        ],
        out_specs=pl.BlockSpec((_T, 1), lambda i: (i, 0)),
        out_shape=jax.ShapeDtypeStruct((_B, 1), jnp.float32),
    )(ue, me, W1, W2, W3, W4, W5, b1, b2, b3, b4, b5,
      g1, be1, g2, be2, g3, be3, g4, be4)


def kernel(users, movies, user_table, movie_table,
           W1, b1, W2, b2, W3, b3, W4, b4, W5, b5,
           g1, be1, g2, be2, g3, be3, g4, be4):
    ue, me = _gather_kernel()(user_table, movie_table, users, movies)
    r = lambda v: v.reshape(1, -1)
    out = _mlp(ue, me, W1, W2, W3, W4, W5,
               r(b1), r(b2), r(b3), r(b4), r(b5),
               r(g1), r(be1), r(g2), r(be2), r(g3), r(be3), r(g4), r(be4))
    return out.reshape(_B)
